# baseline (speedup 1.0000x reference)
import functools

import jax
import jax.numpy as jnp
from jax import lax
from jax.experimental import pallas as pl
from jax.experimental.pallas import tpu as pltpu
from jax.experimental.pallas import tpu_sc as plsc

D = 64
S = 200
NC = 2
NS = 16
NW = NC * NS
BBLK = 128


def _body(idx_hbm, pos_hbm, table_hbm, out_hbm,
          idx_v, pos_v, gb0, gb1, ob0, ob1, pi0, pi1,
          gsem0, gsem1, osem0, osem1):
    gbuf = (gb0, gb1)
    obuf = (ob0, ob1)
    pidx = (pi0, pi1)
    gsem = (gsem0, gsem1)
    osem = (osem0, osem1)

    wid = lax.axis_index("sub") * NC + lax.axis_index("core")
    b0 = wid * BBLK

    pltpu.sync_copy(idx_hbm.at[:, pl.ds(b0, BBLK)], idx_v)
    pltpu.sync_copy(pos_hbm, pos_v)

    iota16 = jnp.arange(16, dtype=jnp.int32)
    zeros16 = jnp.zeros((16,), jnp.int32)

    def compute_pidx(s, b):
        for j in range(BBLK // 16):
            v = idx_v[s, pl.ds(16 * j, 16)]
            pidx[b][pl.ds(16 * j, 16)] = lax.shift_right_logical(v, 1)

    def gather_start(s, b):
        pltpu.async_copy(table_hbm.at[pidx[b]], gbuf[b], gsem[b])

    def gather_wait(b):
        pltpu.make_async_copy(table_hbm.at[pidx[b]], gbuf[b], gsem[b]).wait()

    def out_start(s, b):
        pltpu.async_copy(obuf[b], out_hbm.at[s, :, pl.ds(b0, BBLK)], osem[b])

    def out_wait(s, b):
        pltpu.make_async_copy(obuf[b], out_hbm.at[s, :, pl.ds(b0, BBLK)],
                              osem[b]).wait()

    def transpose_add(s, b):
        gb = gbuf[b]
        ob = obuf[b]
        h64 = []
        for j in range(BBLK // 16):
            v = idx_v[s, pl.ds(16 * j, 16)]
            h64.append(lax.shift_left(jnp.bitwise_and(v, 1), 6))
        s_splat = zeros16 + s

        @plsc.parallel_loop(0, D, unroll=2)
        def _(d):
            d_splat = zeros16 + d
            pos_b = plsc.load_gather(pos_v, [s_splat, d_splat])
            for j in range(BBLK // 16):
                col = h64[j] + d
                val = plsc.load_gather(gb, [iota16 + (16 * j), col])
                ob[d, pl.ds(16 * j, 16)] = val + pos_b

    compute_pidx(0, 0)
    gather_start(0, 0)

    def step(g, carry):
        for b in range(2):
            s = g * 2 + b
            gather_wait(b)

            @pl.when(s + 1 < S)
            def _():
                compute_pidx(s + 1, 1 - b)
                gather_start(s + 1, 1 - b)

            @pl.when(s >= 2)
            def _():
                out_wait(s - 2, b)

            transpose_add(s, b)
            out_start(s, b)
        return carry

    lax.fori_loop(0, S // 2, step, 0)
    out_wait(S - 2, 0)
    out_wait(S - 1, 1)


def kernel(output, word_table, pos_table):
    batch, seq = output.shape
    vocab, d = word_table.shape
    assert batch == NW * BBLK and d == D and seq == S

    idx_t = output.T.astype(jnp.int32)
    pos_pad = jnp.pad(pos_table, ((0, 0), (0, 128 - D)))
    wt_pairs = word_table.reshape(vocab // 2, 2 * D)

    mesh = plsc.VectorSubcoreMesh(core_axis_name="core", subcore_axis_name="sub")
    k = functools.partial(
        pl.kernel,
        mesh=mesh,
        out_type=jax.ShapeDtypeStruct((S, D, batch), jnp.float32),
        scratch_types=[
            pltpu.VMEM((S, BBLK), jnp.int32),
            pltpu.VMEM((S, 128), jnp.float32),
            pltpu.VMEM((BBLK, 2 * D), jnp.float32),
            pltpu.VMEM((BBLK, 2 * D), jnp.float32),
            pltpu.VMEM((D, BBLK), jnp.float32),
            pltpu.VMEM((D, BBLK), jnp.float32),
            pltpu.VMEM((BBLK,), jnp.int32),
            pltpu.VMEM((BBLK,), jnp.int32),
            pltpu.SemaphoreType.DMA,
            pltpu.SemaphoreType.DMA,
            pltpu.SemaphoreType.DMA,
            pltpu.SemaphoreType.DMA,
        ],
        compiler_params=pltpu.CompilerParams(use_tc_tiling_on_sc=True,
                                             needs_layout_passes=False),
    )(_body)

    out_t = k(idx_t, pos_pad, wt_pairs)
    return jnp.transpose(out_t, (2, 0, 1))


# speedup vs baseline: 1.0057x; 1.0057x over previous
import functools

import jax
import jax.numpy as jnp
from jax import lax
from jax.experimental import pallas as pl
from jax.experimental.pallas import tpu as pltpu
from jax.experimental.pallas import tpu_sc as plsc

D = 64
S = 200
NC = 2
NS = 16
NW = NC * NS
BBLK = 128


def _body(idx_hbm, pos_hbm, table_hbm, out_hbm,
          idx_v, pos_v, gb0, gb1, ob0, ob1, pi0, pi1,
          gsem0, gsem1, osem0, osem1):
    gbuf = (gb0, gb1)
    obuf = (ob0, ob1)
    pidx = (pi0, pi1)
    gsem = (gsem0, gsem1)
    osem = (osem0, osem1)

    wid = lax.axis_index("sub") * NC + lax.axis_index("core")
    b0 = wid * BBLK

    pltpu.sync_copy(idx_hbm.at[:, pl.ds(b0, BBLK)], idx_v)
    pltpu.sync_copy(pos_hbm, pos_v)

    iota16 = jnp.arange(16, dtype=jnp.int32)
    zeros16 = jnp.zeros((16,), jnp.int32)

    def compute_pidx(s, b):
        for j in range(BBLK // 16):
            v = idx_v[s, pl.ds(16 * j, 16)]
            pidx[b][pl.ds(16 * j, 16)] = lax.shift_right_logical(v, 1)

    def gather_start(s, b):
        pltpu.async_copy(table_hbm.at[pidx[b]], gbuf[b], gsem[b])

    def gather_wait(b):
        pltpu.make_async_copy(table_hbm.at[pidx[b]], gbuf[b], gsem[b]).wait()

    def out_start(s, b):
        pltpu.async_copy(obuf[b], out_hbm.at[s, :, pl.ds(b0, BBLK)], osem[b])

    def out_wait(s, b):
        pltpu.make_async_copy(obuf[b], out_hbm.at[s, :, pl.ds(b0, BBLK)],
                              osem[b]).wait()

    def transpose_add(s, b):
        gb = gbuf[b]
        ob = obuf[b]
        h64 = []
        for j in range(BBLK // 16):
            v = idx_v[s, pl.ds(16 * j, 16)]
            h64.append(lax.shift_left(jnp.bitwise_and(v, 1), 6))
        s_splat = zeros16 + s

        @plsc.parallel_loop(0, D, unroll=8)
        def _(d):
            d_splat = zeros16 + d
            pos_b = plsc.load_gather(pos_v, [s_splat, d_splat])
            for j in range(BBLK // 16):
                col = h64[j] + d
                val = plsc.load_gather(gb, [iota16 + (16 * j), col])
                ob[d, pl.ds(16 * j, 16)] = val + pos_b

    compute_pidx(0, 0)
    gather_start(0, 0)

    def step(g, carry):
        for b in range(2):
            s = g * 2 + b
            gather_wait(b)

            @pl.when(s + 1 < S)
            def _():
                compute_pidx(s + 1, 1 - b)
                gather_start(s + 1, 1 - b)

            @pl.when(s >= 2)
            def _():
                out_wait(s - 2, b)

            transpose_add(s, b)
            out_start(s, b)
        return carry

    lax.fori_loop(0, S // 2, step, 0)
    out_wait(S - 2, 0)
    out_wait(S - 1, 1)


def kernel(output, word_table, pos_table):
    batch, seq = output.shape
    vocab, d = word_table.shape
    assert batch == NW * BBLK and d == D and seq == S

    idx_t = output.T.astype(jnp.int32)
    pos_pad = jnp.pad(pos_table, ((0, 0), (0, 128 - D)))
    wt_pairs = word_table.reshape(vocab // 2, 2 * D)

    mesh = plsc.VectorSubcoreMesh(core_axis_name="core", subcore_axis_name="sub")
    k = functools.partial(
        pl.kernel,
        mesh=mesh,
        out_type=jax.ShapeDtypeStruct((S, D, batch), jnp.float32),
        scratch_types=[
            pltpu.VMEM((S, BBLK), jnp.int32),
            pltpu.VMEM((S, 128), jnp.float32),
            pltpu.VMEM((BBLK, 2 * D), jnp.float32),
            pltpu.VMEM((BBLK, 2 * D), jnp.float32),
            pltpu.VMEM((D, BBLK), jnp.float32),
            pltpu.VMEM((D, BBLK), jnp.float32),
            pltpu.VMEM((BBLK,), jnp.int32),
            pltpu.VMEM((BBLK,), jnp.int32),
            pltpu.SemaphoreType.DMA,
            pltpu.SemaphoreType.DMA,
            pltpu.SemaphoreType.DMA,
            pltpu.SemaphoreType.DMA,
        ],
        compiler_params=pltpu.CompilerParams(use_tc_tiling_on_sc=True,
                                             needs_layout_passes=False),
    )(_body)

    out_t = k(idx_t, pos_pad, wt_pairs)
    return jnp.transpose(out_t, (2, 0, 1))


# speedup vs baseline: 1.3831x; 1.3753x over previous
import functools

import jax
import jax.numpy as jnp
from jax import lax
from jax.experimental import pallas as pl
from jax.experimental.pallas import tpu as pltpu
from jax.experimental.pallas import tpu_sc as plsc

D = 64
S = 200
NC = 2
NS = 16
NW = NC * NS
BBLK = 128


def _body(idx_hbm, pos_hbm, table_hbm, out_hbm,
          idx_v, pos_v, gb0, gb1, ob0, ob1, pi0, pi1,
          gsem0, gsem1, osem0, osem1):
    gbuf = (gb0, gb1)
    obuf = (ob0, ob1)
    pidx = (pi0, pi1)
    gsem = (gsem0, gsem1)
    osem = (osem0, osem1)

    wid = lax.axis_index("sub") * NC + lax.axis_index("core")
    b0 = wid * BBLK

    pltpu.sync_copy(idx_hbm.at[:, pl.ds(b0, BBLK)], idx_v)
    pltpu.sync_copy(pos_hbm, pos_v)

    iota16 = jnp.arange(16, dtype=jnp.int32)
    zeros16 = jnp.zeros((16,), jnp.int32)

    def compute_pidx(s, b):
        for j in range(BBLK // 16):
            v = idx_v[s, pl.ds(16 * j, 16)]
            pidx[b][pl.ds(16 * j, 16)] = lax.shift_right_logical(v, 1)

    def gather_start(s, b):
        pltpu.async_copy(table_hbm.at[pidx[b]], gbuf[b], gsem[b])

    def gather_wait(b):
        pltpu.make_async_copy(table_hbm.at[pidx[b]], gbuf[b], gsem[b]).wait()

    def out_start(s, b):
        pltpu.async_copy(obuf[b], out_hbm.at[s, :, pl.ds(b0, BBLK)], osem[b])

    def out_wait(s, b):
        pltpu.make_async_copy(obuf[b], out_hbm.at[s, :, pl.ds(b0, BBLK)],
                              osem[b]).wait()

    def transpose_add(s, b):
        gb = gbuf[b]
        ob = obuf[b]
        h64 = []
        rowj = []
        for j in range(BBLK // 16):
            v = idx_v[s, pl.ds(16 * j, 16)]
            h64.append(lax.shift_left(jnp.bitwise_and(v, 1), 6))
            rowj.append(iota16 + 16 * j)
        s_splat = zeros16 + s

        for d0 in range(0, D, 16):
            @plsc.parallel_loop(0, 16, unroll=4)
            def _(m):
                dm = jnp.bitwise_and(iota16 + m, 15) + d0
                pos_m = plsc.load_gather(pos_v, [s_splat, dm])
                for j in range(BBLK // 16):
                    cols = h64[j] + dm
                    val = plsc.load_gather(gb, [rowj[j], cols])
                    plsc.store_scatter(ob, [dm, rowj[j]], val + pos_m)

    compute_pidx(0, 0)
    gather_start(0, 0)

    def step(g, carry):
        for b in range(2):
            s = g * 2 + b
            gather_wait(b)

            @pl.when(s + 1 < S)
            def _():
                compute_pidx(s + 1, 1 - b)
                gather_start(s + 1, 1 - b)

            @pl.when(s >= 2)
            def _():
                out_wait(s - 2, b)

            transpose_add(s, b)
            out_start(s, b)
        return carry

    lax.fori_loop(0, S // 2, step, 0)
    out_wait(S - 2, 0)
    out_wait(S - 1, 1)


def kernel(output, word_table, pos_table):
    batch, seq = output.shape
    vocab, d = word_table.shape
    assert batch == NW * BBLK and d == D and seq == S

    idx_t = output.T.astype(jnp.int32)
    pos_pad = jnp.pad(pos_table, ((0, 0), (0, 128 - D)))
    wt_pairs = word_table.reshape(vocab // 2, 2 * D)

    mesh = plsc.VectorSubcoreMesh(core_axis_name="core", subcore_axis_name="sub")
    k = functools.partial(
        pl.kernel,
        mesh=mesh,
        out_type=jax.ShapeDtypeStruct((S, D, batch), jnp.float32),
        scratch_types=[
            pltpu.VMEM((S, BBLK), jnp.int32),
            pltpu.VMEM((S, 128), jnp.float32),
            pltpu.VMEM((BBLK, 2 * D), jnp.float32),
            pltpu.VMEM((BBLK, 2 * D), jnp.float32),
            pltpu.VMEM((D, BBLK), jnp.float32),
            pltpu.VMEM((D, BBLK), jnp.float32),
            pltpu.VMEM((BBLK,), jnp.int32),
            pltpu.VMEM((BBLK,), jnp.int32),
            pltpu.SemaphoreType.DMA,
            pltpu.SemaphoreType.DMA,
            pltpu.SemaphoreType.DMA,
            pltpu.SemaphoreType.DMA,
        ],
        compiler_params=pltpu.CompilerParams(use_tc_tiling_on_sc=True,
                                             needs_layout_passes=False),
    )(_body)

    out_t = k(idx_t, pos_pad, wt_pairs)
    return jnp.transpose(out_t, (2, 0, 1))
